# baseline (speedup 1.0000x reference)
import functools

import jax
import jax.numpy as jnp
from jax import lax
from jax.experimental import pallas as pl
from jax.experimental.pallas import tpu as pltpu
from jax.experimental.pallas import tpu_sc as plsc

DIM = 128
GAMMA = 12.0
EPSILON = 2.0
PI = 3.141592653589793
EMB_RANGE = (GAMMA + EPSILON) / DIM
PHASE_SCALE = PI / EMB_RANGE

B = 16384
NC = 2
NS = 16
NW = NC * NS
SPW = B // NW
CHUNK = 128
CPW = SPW // CHUNK

TWO_PI = 6.283185307179586
INV_2PI = 1.0 / TWO_PI
MAGIC = 12582912.0

SIN_C = (9.99450173e-01, -1.65838429e-01, 7.99857532e-03, -1.47740438e-04)
COS_C = (9.98987133e-01, -4.96248575e-01, 3.95222834e-02, -9.92859893e-04)


def _poly(coeffs, x):
    y = jnp.full((16,), coeffs[-1], dtype=jnp.float32)
    for c in coeffs[-2::-1]:
        y = y * x + c
    return y


def _take(v, idx):
    return v.at[idx].get(mode="promise_in_bounds")


@functools.cache
def _make_sc_score():
    mesh = plsc.VectorSubcoreMesh(core_axis_name="c", subcore_axis_name="s")

    @functools.partial(
        pl.kernel,
        mesh=mesh,
        out_type=jax.ShapeDtypeStruct((B,), jnp.float32),
        scratch_types=[
            pltpu.VMEM((SPW * 3,), jnp.int32),
            pltpu.VMEM((SPW,), jnp.int32),
            pltpu.VMEM((SPW,), jnp.int32),
            pltpu.VMEM((SPW,), jnp.int32),
            pltpu.VMEM((SPW + 16,), jnp.int32),
            pltpu.VMEM((CHUNK, 128), jnp.float32),
            pltpu.VMEM((CHUNK, 128), jnp.float32),
            pltpu.VMEM((CHUNK, 128), jnp.float32),
            pltpu.VMEM((CHUNK, 128), jnp.float32),
            pltpu.VMEM((CHUNK, 128), jnp.float32),
            pltpu.VMEM((CHUNK, 128), jnp.float32),
            pltpu.VMEM((CHUNK, 16), jnp.float32),
            pltpu.VMEM((SPW,), jnp.float32),
            pltpu.SemaphoreType.DMA,
            pltpu.SemaphoreType.DMA,
            pltpu.SemaphoreType.DMA,
            pltpu.SemaphoreType.DMA,
            pltpu.SemaphoreType.DMA,
            pltpu.SemaphoreType.DMA,
            pltpu.SemaphoreType.DMA,
        ],
    )
    def _sc_score(samp_hbm, ent_hbm, rel_hbm, out_hbm,
                  samp_v, ih_v, ir_v, it_v, par_v,
                  h0, h1, r0, r1, t0, t1, sc_w, sc_v,
                  sh0, sh1, sr0, sr1, st0, st1, sidx):
        h_bufs, r_bufs, t_bufs = (h0, h1), (r0, r1), (t0, t1)
        h_sems, r_sems, t_sems = (sh0, sh1), (sr0, sr1), (st0, st1)

        wid = lax.axis_index("s") * NC + lax.axis_index("c")
        base = wid * SPW
        pltpu.async_copy(samp_hbm.at[pl.ds(base * 3, SPW * 3)],
                         samp_v, sidx).wait()

        iota = lax.iota(jnp.int32, 16)
        lane_sel = [iota == i for i in range(16)]

        cols = []
        for colid in range(3):
            srcpos = iota * 3 + colid
            cols.append((srcpos & 15, srcpos >> 4))
        for v in range(SPW // 16):
            f = v * 48
            regs = [samp_v[pl.ds(f + 16 * i, 16)] for i in range(3)]
            hrt = []
            for lane_idx, bank in cols:
                g0 = _take(regs[0], lane_idx)
                g1 = _take(regs[1], lane_idx)
                g2 = _take(regs[2], lane_idx)
                hrt.append(jnp.where(bank == 2, g2,
                                     jnp.where(bank == 1, g1, g0)))
            hv, rv, tv = hrt
            sl16 = pl.ds(v * 16, 16)
            ih_v[sl16] = hv
            ir_v[sl16] = rv >> 1
            it_v[sl16] = tv
            par_v[sl16] = (rv & 1) << 6

        def fire(c):
            p = c % 2
            sl = pl.ds(c * CHUNK, CHUNK)
            return (
                pltpu.async_copy(ent_hbm.at[ih_v.at[sl]], h_bufs[p], h_sems[p]),
                pltpu.async_copy(rel_hbm.at[ir_v.at[sl]], r_bufs[p], r_sems[p]),
                pltpu.async_copy(ent_hbm.at[it_v.at[sl]], t_bufs[p], t_sems[p]),
            )

        pending = fire(0)
        for c in range(CPW):
            p = c % 2
            for d in pending:
                d.wait()
            if c + 1 < CPW:
                pending = fire(c + 1)
            hb, rb, tb = h_bufs[p], r_bufs[p], t_bufs[p]

            def one_sample(s, po, hb=hb, rb=rb, tb=tb):
                macc = jnp.zeros((16,), jnp.float32)
                for k in range(4):
                    re_h = hb[s, pl.ds(16 * k, 16)]
                    im_h = hb[s, pl.ds(64 + 16 * k, 16)]
                    re_t = tb[s, pl.ds(16 * k, 16)]
                    im_t = tb[s, pl.ds(64 + 16 * k, 16)]
                    rl = rb[s, pl.ds(po + 16 * k, 16)]
                    ph = rl * PHASE_SCALE
                    kf = (ph * INV_2PI + MAGIC) - MAGIC
                    r = ph - kf * TWO_PI
                    r2 = r * r
                    sn = r * _poly(SIN_C, r2)
                    co = _poly(COS_C, r2)
                    re_sc = re_h * co - im_h * sn - re_t
                    im_sc = re_h * sn + im_h * co - im_t
                    m2 = re_sc * re_sc + im_sc * im_sc
                    yi = 0x5F3759DF - (lax.bitcast_convert_type(m2, jnp.int32) >> 1)
                    y = lax.bitcast_convert_type(yi, jnp.float32)
                    y = y * (1.5 - 0.5 * (m2 * y) * y)
                    macc = macc + m2 * y
                for sh in (8, 4, 2, 1):
                    macc = macc + _take(macc, iota ^ sh)
                sc_w[s, :] = GAMMA - macc

            def sample_body(i, carry, c=c, one_sample=one_sample):
                povec = par_v[pl.ds(c * CHUNK + i * 4, 16)]
                for u in range(4):
                    one_sample(i * 4 + u, povec[u])
                return carry

            lax.fori_loop(0, CHUNK // 4, sample_body, 0)

            for g in range(8):
                acc = sc_w[g * 16, :]
                for i in range(1, 16):
                    acc = jnp.where(lane_sel[i], sc_w[g * 16 + i, :], acc)
                sc_v[pl.ds(c * CHUNK + g * 16, 16)] = acc

        pltpu.sync_copy(sc_v, out_hbm.at[pl.ds(base, SPW)])

    return _sc_score


def kernel(sample, ent_emb, rel_emb):
    samp = sample.astype(jnp.int32).reshape(B * 3)
    rel2 = rel_emb.reshape(rel_emb.shape[0] // 2, 128)
    out = _make_sc_score()(samp, ent_emb, rel2)
    return out.reshape(B, 1)


# speedup vs baseline: 1.0181x; 1.0181x over previous
import functools

import jax
import jax.numpy as jnp
from jax import lax
from jax.experimental import pallas as pl
from jax.experimental.pallas import tpu as pltpu
from jax.experimental.pallas import tpu_sc as plsc

DIM = 128
GAMMA = 12.0
EPSILON = 2.0
PI = 3.141592653589793
EMB_RANGE = (GAMMA + EPSILON) / DIM
PHASE_SCALE = PI / EMB_RANGE

B = 16384
NC = 2
NS = 16
NW = NC * NS
SPW = B // NW
CHUNK = 128
CPW = SPW // CHUNK

TWO_PI = 6.283185307179586
INV_2PI = 1.0 / TWO_PI
MAGIC = 12582912.0

SIN_C = (9.99450173e-01, -1.65838429e-01, 7.99857532e-03, -1.47740438e-04)
COS_C = (9.98987133e-01, -4.96248575e-01, 3.95222834e-02, -9.92859893e-04)


def _poly(coeffs, x):
    y = jnp.full((16,), coeffs[-1], dtype=jnp.float32)
    for c in coeffs[-2::-1]:
        y = y * x + c
    return y


def _take(v, idx):
    return v.at[idx].get(mode="promise_in_bounds")


@functools.cache
def _make_sc_score():
    mesh = plsc.VectorSubcoreMesh(core_axis_name="c", subcore_axis_name="s")

    @functools.partial(
        pl.kernel,
        mesh=mesh,
        out_type=jax.ShapeDtypeStruct((B,), jnp.float32),
        scratch_types=[
            pltpu.VMEM((SPW,), jnp.int32),
            pltpu.VMEM((SPW,), jnp.int32),
            pltpu.VMEM((SPW,), jnp.int32),
            pltpu.VMEM((SPW + 16,), jnp.int32),
            pltpu.VMEM((CHUNK, 128), jnp.float32),
            pltpu.VMEM((CHUNK, 128), jnp.float32),
            pltpu.VMEM((CHUNK, 128), jnp.float32),
            pltpu.VMEM((CHUNK, 128), jnp.float32),
            pltpu.VMEM((CHUNK, 128), jnp.float32),
            pltpu.VMEM((CHUNK, 128), jnp.float32),
            pltpu.VMEM((CHUNK, 16), jnp.float32),
            pltpu.VMEM((SPW,), jnp.float32),
            pltpu.SemaphoreType.DMA,
            pltpu.SemaphoreType.DMA,
            pltpu.SemaphoreType.DMA,
            pltpu.SemaphoreType.DMA,
            pltpu.SemaphoreType.DMA,
            pltpu.SemaphoreType.DMA,
            pltpu.SemaphoreType.DMA,
        ],
    )
    def _sc_score(ih_hbm, ir_hbm, it_hbm, par_hbm, ent_hbm, rel_hbm, out_hbm,
                  ih_v, ir_v, it_v, par_v,
                  h0, h1, r0, r1, t0, t1, sc_w, sc_v,
                  sh0, sh1, sr0, sr1, st0, st1, sidx):
        h_bufs, r_bufs, t_bufs = (h0, h1), (r0, r1), (t0, t1)
        h_sems, r_sems, t_sems = (sh0, sh1), (sr0, sr1), (st0, st1)

        wid = lax.axis_index("s") * NC + lax.axis_index("c")
        base = wid * SPW
        pltpu.sync_copy(ih_hbm.at[pl.ds(base, SPW)], ih_v)
        pltpu.sync_copy(ir_hbm.at[pl.ds(base, SPW)], ir_v)
        pltpu.sync_copy(it_hbm.at[pl.ds(base, SPW)], it_v)
        pltpu.async_copy(par_hbm.at[pl.ds(base, SPW)],
                         par_v.at[pl.ds(0, SPW)], sidx).wait()

        iota = lax.iota(jnp.int32, 16)
        lane_sel = [iota == i for i in range(16)]

        def fire(c):
            p = c % 2
            sl = pl.ds(c * CHUNK, CHUNK)
            return (
                pltpu.async_copy(ent_hbm.at[ih_v.at[sl]], h_bufs[p], h_sems[p]),
                pltpu.async_copy(rel_hbm.at[ir_v.at[sl]], r_bufs[p], r_sems[p]),
                pltpu.async_copy(ent_hbm.at[it_v.at[sl]], t_bufs[p], t_sems[p]),
            )

        pending = fire(0)
        for c in range(CPW):
            p = c % 2
            for d in pending:
                d.wait()
            if c + 1 < CPW:
                pending = fire(c + 1)
            hb, rb, tb = h_bufs[p], r_bufs[p], t_bufs[p]

            def one_sample(s, po, hb=hb, rb=rb, tb=tb):
                macc = jnp.zeros((16,), jnp.float32)
                for k in range(4):
                    re_h = hb[s, pl.ds(16 * k, 16)]
                    im_h = hb[s, pl.ds(64 + 16 * k, 16)]
                    re_t = tb[s, pl.ds(16 * k, 16)]
                    im_t = tb[s, pl.ds(64 + 16 * k, 16)]
                    rl = rb[s, pl.ds(po + 16 * k, 16)]
                    ph = rl * PHASE_SCALE
                    kf = (ph * INV_2PI + MAGIC) - MAGIC
                    r = ph - kf * TWO_PI
                    r2 = r * r
                    sn = r * _poly(SIN_C, r2)
                    co = _poly(COS_C, r2)
                    re_sc = re_h * co - im_h * sn - re_t
                    im_sc = re_h * sn + im_h * co - im_t
                    m2 = re_sc * re_sc + im_sc * im_sc
                    yi = 0x5F3759DF - (lax.bitcast_convert_type(m2, jnp.int32) >> 1)
                    y = lax.bitcast_convert_type(yi, jnp.float32)
                    y = y * (1.5 - 0.5 * (m2 * y) * y)
                    macc = macc + m2 * y
                for sh in (8, 4, 2, 1):
                    macc = macc + _take(macc, iota ^ sh)
                sc_w[s, :] = GAMMA - macc

            def sample_body(i, carry, c=c, one_sample=one_sample):
                povec = par_v[pl.ds(c * CHUNK + i * 8, 16)]
                for u in range(8):
                    one_sample(i * 8 + u, povec[u])
                return carry

            lax.fori_loop(0, CHUNK // 8, sample_body, 0)

            for g in range(8):
                acc = sc_w[g * 16, :]
                for i in range(1, 16):
                    acc = jnp.where(lane_sel[i], sc_w[g * 16 + i, :], acc)
                sc_v[pl.ds(c * CHUNK + g * 16, 16)] = acc

        pltpu.sync_copy(sc_v, out_hbm.at[pl.ds(base, SPW)])

    return _sc_score


def kernel(sample, ent_emb, rel_emb):
    s32 = sample.astype(jnp.int32)
    n_ent = ent_emb.shape[0]
    ih = jnp.clip(s32[:, 0], 0, n_ent - 1)
    ir = s32[:, 1] >> 1
    it = jnp.clip(s32[:, 2], 0, n_ent - 1)
    par = (s32[:, 1] & 1) << 6
    rel2 = rel_emb.reshape(rel_emb.shape[0] // 2, 128)
    out = _make_sc_score()(ih, ir, it, par, ent_emb, rel2)
    return out.reshape(B, 1)


# speedup vs baseline: 1.0365x; 1.0182x over previous
import functools

import jax
import jax.numpy as jnp
from jax import lax
from jax.experimental import pallas as pl
from jax.experimental.pallas import tpu as pltpu
from jax.experimental.pallas import tpu_sc as plsc

DIM = 128
GAMMA = 12.0
EPSILON = 2.0
PI = 3.141592653589793
EMB_RANGE = (GAMMA + EPSILON) / DIM
PHASE_SCALE = PI / EMB_RANGE

B = 16384
NC = 2
NS = 16
NW = NC * NS
SPW = B // NW
CHUNK = 128
CPW = SPW // CHUNK

TWO_PI = 6.283185307179586
INV_2PI = 1.0 / TWO_PI
MAGIC = 12582912.0

SIN_C = (9.99450173e-01, -1.65838429e-01, 7.99857532e-03, -1.47740438e-04)
COS_C = (9.98987133e-01, -4.96248575e-01, 3.95222834e-02, -9.92859893e-04)


def _poly(coeffs, x):
    y = jnp.full((16,), coeffs[-1], dtype=jnp.float32)
    for c in coeffs[-2::-1]:
        y = y * x + c
    return y


def _take(v, idx):
    return v.at[idx].get(mode="promise_in_bounds")


@functools.cache
def _make_sc_score():
    mesh = plsc.VectorSubcoreMesh(core_axis_name="c", subcore_axis_name="s")

    @functools.partial(
        pl.kernel,
        mesh=mesh,
        out_type=jax.ShapeDtypeStruct((B,), jnp.float32),
        scratch_types=[
            pltpu.VMEM((SPW,), jnp.int32),
            pltpu.VMEM((SPW,), jnp.int32),
            pltpu.VMEM((SPW,), jnp.int32),
            pltpu.VMEM((SPW + 16,), jnp.int32),
            pltpu.VMEM((CHUNK, 128), jnp.float32),
            pltpu.VMEM((CHUNK, 128), jnp.float32),
            pltpu.VMEM((CHUNK, 128), jnp.float32),
            pltpu.VMEM((CHUNK, 128), jnp.float32),
            pltpu.VMEM((CHUNK, 128), jnp.float32),
            pltpu.VMEM((CHUNK, 128), jnp.float32),
            pltpu.VMEM((CHUNK, 16), jnp.float32),
            pltpu.VMEM((SPW,), jnp.float32),
            pltpu.SemaphoreType.DMA,
            pltpu.SemaphoreType.DMA,
            pltpu.SemaphoreType.DMA,
            pltpu.SemaphoreType.DMA,
            pltpu.SemaphoreType.DMA,
            pltpu.SemaphoreType.DMA,
            pltpu.SemaphoreType.DMA,
        ],
    )
    def _sc_score(ih_hbm, ir_hbm, it_hbm, par_hbm, ent_hbm, rel_hbm, out_hbm,
                  ih_v, ir_v, it_v, par_v,
                  h0, h1, r0, r1, t0, t1, sc_w, sc_v,
                  sh0, sh1, sr0, sr1, st0, st1, sidx):
        h_bufs, r_bufs, t_bufs = (h0, h1), (r0, r1), (t0, t1)
        h_sems, r_sems, t_sems = (sh0, sh1), (sr0, sr1), (st0, st1)

        wid = lax.axis_index("s") * NC + lax.axis_index("c")
        base = wid * SPW
        pltpu.sync_copy(ih_hbm.at[pl.ds(base, SPW)], ih_v)
        pltpu.sync_copy(ir_hbm.at[pl.ds(base, SPW)], ir_v)
        pltpu.sync_copy(it_hbm.at[pl.ds(base, SPW)], it_v)
        pltpu.async_copy(par_hbm.at[pl.ds(base, SPW)],
                         par_v.at[pl.ds(0, SPW)], sidx).wait()

        iota = lax.iota(jnp.int32, 16)
        lane_sel = [iota == i for i in range(16)]

        def fire(c):
            p = c % 2
            sl = pl.ds(c * CHUNK, CHUNK)
            return (
                pltpu.async_copy(ent_hbm.at[ih_v.at[sl]], h_bufs[p], h_sems[p]),
                pltpu.async_copy(rel_hbm.at[ir_v.at[sl]], r_bufs[p], r_sems[p]),
                pltpu.async_copy(ent_hbm.at[it_v.at[sl]], t_bufs[p], t_sems[p]),
            )

        pending = fire(0)
        for c in range(CPW):
            p = c % 2
            for d in pending:
                d.wait()
            if c + 1 < CPW:
                pending = fire(c + 1)
            hb, rb, tb = h_bufs[p], r_bufs[p], t_bufs[p]

            def one_sample(s, po, hb=hb, rb=rb, tb=tb):
                macc = jnp.zeros((16,), jnp.float32)
                for k in range(4):
                    re_h = hb[s, pl.ds(16 * k, 16)]
                    im_h = hb[s, pl.ds(64 + 16 * k, 16)]
                    re_t = tb[s, pl.ds(16 * k, 16)]
                    im_t = tb[s, pl.ds(64 + 16 * k, 16)]
                    rl = rb[s, pl.ds(po + 16 * k, 16)]
                    ph = rl * PHASE_SCALE
                    kf = (ph * INV_2PI + MAGIC) - MAGIC
                    r = ph - kf * TWO_PI
                    r2 = r * r
                    sn = r * _poly(SIN_C, r2)
                    co = _poly(COS_C, r2)
                    re_sc = re_h * co - im_h * sn - re_t
                    im_sc = re_h * sn + im_h * co - im_t
                    m2 = re_sc * re_sc + im_sc * im_sc
                    yi = 0x5F3759DF - (lax.bitcast_convert_type(m2, jnp.int32) >> 1)
                    y = lax.bitcast_convert_type(yi, jnp.float32)
                    y = y * (1.5 - 0.5 * (m2 * y) * y)
                    macc = macc + m2 * y
                for sh in (8, 4, 2, 1):
                    macc = macc + _take(macc, iota ^ sh)
                sc_w[s, :] = GAMMA - macc

            def sample_body(i, carry, c=c, one_sample=one_sample):
                povec = par_v[pl.ds(c * CHUNK + i * 4, 16)]
                for u in range(4):
                    one_sample(i * 4 + u, povec[u])
                return carry

            lax.fori_loop(0, CHUNK // 4, sample_body, 0)

            for g in range(8):
                acc = sc_w[g * 16, :]
                for i in range(1, 16):
                    acc = jnp.where(lane_sel[i], sc_w[g * 16 + i, :], acc)
                sc_v[pl.ds(c * CHUNK + g * 16, 16)] = acc

        pltpu.sync_copy(sc_v, out_hbm.at[pl.ds(base, SPW)])

    return _sc_score


def kernel(sample, ent_emb, rel_emb):
    s32 = sample.astype(jnp.int32)
    n_ent = ent_emb.shape[0]
    ih = jnp.clip(s32[:, 0], 0, n_ent - 1)
    ir = s32[:, 1] >> 1
    it = jnp.clip(s32[:, 2], 0, n_ent - 1)
    par = (s32[:, 1] & 1) << 6
    rel2 = rel_emb.reshape(rel_emb.shape[0] // 2, 128)
    out = _make_sc_score()(ih, ir, it, par, ent_emb, rel2)
    return out.reshape(B, 1)
